# Initial kernel scaffold; baseline (speedup 1.0000x reference)
import jax
import jax.numpy as jnp
from jax.experimental import pallas as pl


def kernel(x, edge_index, W0, b0, W1, b1, W2, b2, W3, b3, Wfc, bfc):
    raise NotImplementedError("write your pallas kernel here")


# speedup vs baseline: 5.2257x; 5.2257x over previous
import dataclasses
import functools

import jax
import jax.numpy as jnp
from jax import lax
from jax.experimental import pallas as pl
from jax.experimental.pallas import tpu as pltpu
from jax.experimental.pallas import tpu_sc as plsc

_NC = 2
_NS = 16
_NW = _NC * _NS
_CH = 80
_ZR = 128
_LANES = 16


def _pad_rows(N):
    per = -(-N // _NS)
    per = -(-per // _ZR) * _ZR
    return per * _NS, per


def _make_sc_aggregate(N, D, E, with_hist):
    mesh = plsc.VectorSubcoreMesh(core_axis_name="c", subcore_axis_name="s")
    epw = E // _NW
    NP, rps = _pad_rows(N)
    assert E % (_NW * _CH) == 0 and rps % _ZR == 0

    out_type = [jax.ShapeDtypeStruct((_NC, NP, D), jnp.float32)]
    scratch = [
        pltpu.VMEM((_CH,), jnp.int32),
        pltpu.VMEM((_CH,), jnp.int32),
        pltpu.VMEM((_CH, D), jnp.float32),
        pltpu.VMEM((_ZR, D), jnp.float32),
        pltpu.VMEM_SHARED((NP, D), jnp.float32),
        pltpu.SemaphoreType.DMA,
    ]
    if with_hist:
        out_type.append(jax.ShapeDtypeStruct((_NW, 1, N), jnp.float32))
        scratch.append(pltpu.VMEM((N,), jnp.float32))

    def body(table, src, dst, part_out, *rest):
        if with_hist:
            hist_out, idx_s, idx_d, rows, zbuf, acc, sem, hist = rest
        else:
            idx_s, idx_d, rows, zbuf, acc, sem = rest
            hist_out = hist = None
        c = lax.axis_index("c")
        s = lax.axis_index("s")
        wid = c * _NS + s
        zv = jnp.zeros((_LANES,), jnp.float32)

        @pl.loop(0, _ZR)
        def _(i):
            for k in range(D // _LANES):
                zbuf[i, pl.ds(k * _LANES, _LANES)] = zv

        for t in range(rps // _ZR):
            pltpu.sync_copy(zbuf, acc.at[pl.ds(s * rps + t * _ZR, _ZR)])

        if with_hist:
            @pl.loop(0, N // _LANES)
            def _(i):
                hist[pl.ds(i * _LANES, _LANES)] = zv

        plsc.subcore_barrier()

        base = wid * epw
        ones = jnp.ones((_LANES,), jnp.float32)

        @pl.loop(0, epw // _CH)
        def _(j):
            off = base + j * _CH
            pltpu.sync_copy(src.at[pl.ds(off, _CH)], idx_s)
            pltpu.sync_copy(dst.at[pl.ds(off, _CH)], idx_d)
            pltpu.async_copy(table.at[idx_s], rows, sem).wait()
            pltpu.sync_copy(rows, acc.at[idx_d], add=True)
            if with_hist:
                for k in range(_CH // _LANES):
                    dv = idx_d[pl.ds(k * _LANES, _LANES)]
                    plsc.addupdate_scatter(hist, [dv], ones)

        plsc.subcore_barrier()
        pltpu.sync_copy(acc.at[pl.ds(s * rps, rps)],
                        part_out.at[c, pl.ds(s * rps, rps)])
        if with_hist:
            pltpu.sync_copy(hist, hist_out.at[wid, 0])

    cp = pltpu.CompilerParams()
    if "needs_layout_passes" in pltpu.CompilerParams.__dataclass_fields__:
        cp = dataclasses.replace(cp, needs_layout_passes=False)
    return pl.kernel(body, out_type=tuple(out_type) if with_hist else out_type[0],
                     mesh=mesh, scratch_types=scratch, compiler_params=cp)


def _deg_body(hist_ref, o_ref):
    cnt = jnp.sum(hist_ref[...], axis=(0, 1))
    o_ref[...] = (1.0 / jnp.maximum(cnt, 1.0))[:, None]


def _tc_inv_deg(hist, N):
    return pl.pallas_call(
        _deg_body,
        out_shape=jax.ShapeDtypeStruct((N, 1), jnp.float32),
    )(hist)


def _layer_body(p_ref, invdeg_ref, x_ref, w_ref, b_ref, o_ref):
    p = p_ref[...]
    agg = (p[0] + p[1]) * invdeg_ref[...]
    z = jnp.dot(agg, w_ref[...], preferred_element_type=jnp.float32) + b_ref[...]
    o_ref[...] = jnp.maximum(z, 0.0) + x_ref[...]


def _final_body(p_ref, invdeg_ref, x_ref, w_ref, b_ref, wfc_ref, bfc_ref, o_ref):
    p = p_ref[...]
    agg = (p[0] + p[1]) * invdeg_ref[...]
    z = jnp.dot(agg, w_ref[...], preferred_element_type=jnp.float32) + b_ref[...]
    h = jnp.maximum(z, 0.0) + x_ref[...]
    o_ref[...] = jnp.dot(h, wfc_ref[...],
                         preferred_element_type=jnp.float32) + bfc_ref[...]


def _tc_layer(part, invdeg, x, W, b, BN=1000):
    N, D = x.shape
    return pl.pallas_call(
        _layer_body,
        grid=(N // BN,),
        in_specs=[
            pl.BlockSpec((_NC, BN, D), lambda i: (0, i, 0)),
            pl.BlockSpec((BN, 1), lambda i: (i, 0)),
            pl.BlockSpec((BN, D), lambda i: (i, 0)),
            pl.BlockSpec((D, D), lambda i: (0, 0)),
            pl.BlockSpec((1, D), lambda i: (0, 0)),
        ],
        out_specs=pl.BlockSpec((BN, D), lambda i: (i, 0)),
        out_shape=jax.ShapeDtypeStruct((N, D), jnp.float32),
    )(part, invdeg, x, W, b.reshape(1, D))


def _tc_final(part, invdeg, x, W, b, Wfc, bfc, BN=1000):
    N, D = x.shape
    OUT = Wfc.shape[1]
    return pl.pallas_call(
        _final_body,
        grid=(N // BN,),
        in_specs=[
            pl.BlockSpec((_NC, BN, D), lambda i: (0, i, 0)),
            pl.BlockSpec((BN, 1), lambda i: (i, 0)),
            pl.BlockSpec((BN, D), lambda i: (i, 0)),
            pl.BlockSpec((D, D), lambda i: (0, 0)),
            pl.BlockSpec((1, D), lambda i: (0, 0)),
            pl.BlockSpec((D, OUT), lambda i: (0, 0)),
            pl.BlockSpec((1, OUT), lambda i: (0, 0)),
        ],
        out_specs=pl.BlockSpec((BN, OUT), lambda i: (i, 0)),
        out_shape=jax.ShapeDtypeStruct((N, OUT), jnp.float32),
    )(part, invdeg, x, W, b.reshape(1, D), Wfc, bfc.reshape(1, OUT))


def kernel(x, edge_index, W0, b0, W1, b1, W2, b2, W3, b3, Wfc, bfc):
    N, D = x.shape
    E = edge_index.shape[1]
    agg_first = _make_sc_aggregate(N, D, E, with_hist=True)
    agg_rest = _make_sc_aggregate(N, D, E, with_hist=False)

    src = edge_index[0]
    dst = edge_index[1]
    part, hist = agg_first(x, src, dst)
    invdeg = _tc_inv_deg(hist, N)
    h = _tc_layer(part, invdeg, x, W0, b0)
    for W, b in ((W1, b1), (W2, b2)):
        part = agg_rest(h, src, dst)
        h = _tc_layer(part, invdeg, x, W, b)
    part = agg_rest(h, src, dst)
    out = _tc_final(part, invdeg, x, W3, b3, Wfc, bfc)
    return out[:, 0]
